# Initial kernel scaffold; baseline (speedup 1.0000x reference)
import jax
import jax.numpy as jnp
from jax.experimental import pallas as pl


def kernel(x, Wqk, bqk, Wv, bv, Wo, bo, R, n1_scale, n1_bias, Wa, ba, Wb, bb, Wc, bc, n2_scale, n2_bias):
    raise NotImplementedError("write your pallas kernel here")


# speedup vs baseline: 4.1258x; 4.1258x over previous
import functools

import jax
import jax.numpy as jnp
from jax import lax
from jax.experimental import pallas as pl
from jax.experimental.pallas import tpu as pltpu
from jax.experimental.pallas import tpu_sc as plsc

D = 1024
DH = 32
H = D // DH
NB = 32
B, N = 4, 2048
CS = N // NB
M = B * H * N
F = 3 * D

ROWS_BLK = 256
NW = 32
RPW = M // NW
JROWS = 128
NJ = RPW // JROWS


def _ln(xb, scale, bias):
    mu = jnp.mean(xb, axis=1, keepdims=True)
    xc = xb - mu
    var = jnp.sum(xc * xc, axis=1, keepdims=True) * (1.0 / (xb.shape[1] - 1))
    return xc * lax.rsqrt(var + 1e-6) * scale + bias


def _k1_body(x_ref, wqk_ref, bqk_ref, wv_ref, bv_ref, s_ref, b_ref,
             qk_ref, v_ref):
    h1 = _ln(x_ref[...], s_ref[...], b_ref[...])
    qk_ref[...] = jnp.dot(h1, wqk_ref[...], preferred_element_type=jnp.float32) + bqk_ref[...]
    v_ref[...] = jnp.dot(h1, wv_ref[...], preferred_element_type=jnp.float32) + bv_ref[...]


def _k1(x2d, Wqk, bqk, Wv, bv, n1s, n1b):
    nsteps = (B * N) // ROWS_BLK
    row_spec = pl.BlockSpec((ROWS_BLK, D), lambda i: (i, 0))
    full = pl.BlockSpec((D, D), lambda i: (0, 0))
    vec = pl.BlockSpec((1, D), lambda i: (0, 0))
    return pl.pallas_call(
        _k1_body,
        grid=(nsteps,),
        in_specs=[row_spec, full, vec, full, vec, vec, vec],
        out_specs=[row_spec, row_spec],
        out_shape=[jax.ShapeDtypeStruct((B * N, D), jnp.float32)] * 2,
    )(x2d, Wqk, bqk.reshape(1, D), Wv, bv.reshape(1, D), n1s, n1b)


def _k2_body(qkv_ref, r_ref, idx_ref):
    qk = qkv_ref[0, :, :DH]
    rot = jnp.dot(qk, r_ref[0], preferred_element_type=jnp.float32)
    conc = jnp.concatenate([rot, -rot], axis=1)
    mx = jnp.max(conc, axis=1, keepdims=True)
    lane = lax.broadcasted_iota(jnp.int32, (N, NB), 1).astype(jnp.float32)
    bucket = jnp.min(jnp.where(conc == mx, lane, float(NB)), axis=1,
                     keepdims=True)
    oh = (bucket == lane).astype(jnp.float32)

    gi = lax.broadcasted_iota(jnp.int32, (128, 128), 0)
    gj = lax.broadcasted_iota(jnp.int32, (128, 128), 1)
    t128 = (gj < gi).astype(jnp.float32)
    blocks = []
    prun = jnp.zeros((1, NB), jnp.float32)
    for g in range(N // 128):
        og = oh[g * 128:(g + 1) * 128, :]
        within = jnp.dot(t128, og, preferred_element_type=jnp.float32, precision=lax.Precision.HIGHEST)
        blocks.append(within + prun)
        prun = prun + jnp.sum(og, axis=0, keepdims=True)
    counts = jnp.concatenate(blocks, axis=0)
    ui = lax.broadcasted_iota(jnp.int32, (NB, NB), 0)
    uj = lax.broadcasted_iota(jnp.int32, (NB, NB), 1)
    u32 = (ui < uj).astype(jnp.float32)
    prefix = jnp.dot(prun, u32, preferred_element_type=jnp.float32, precision=lax.Precision.HIGHEST)
    rank = jnp.sum((counts + prefix) * oh, axis=1, keepdims=True)

    bh = pl.program_id(0) * H + pl.program_id(1)
    rankg = rank + jnp.float32(bh * N)
    ri = lax.broadcasted_iota(jnp.int32, (N, 128), 0)
    rj = lax.broadcasted_iota(jnp.int32, (N, 128), 1)
    spread = rankg * (lax.rem(ri, 128) == rj).astype(jnp.float32)
    ai = lax.broadcasted_iota(jnp.int32, (N // 128, N), 0)
    aj = lax.broadcasted_iota(jnp.int32, (N // 128, N), 1)
    asel = (lax.div(aj, 128) == ai).astype(jnp.float32)
    idx_ref[...] = jnp.dot(asel, spread,
                           preferred_element_type=jnp.float32, precision=lax.Precision.HIGHEST).astype(jnp.int32)


def _k2(qkv_t, R):
    return pl.pallas_call(
        _k2_body,
        grid=(B, H),
        in_specs=[
            pl.BlockSpec((1, N, 4 * DH), lambda b, h: (b * H + h, 0, 0)),
            pl.BlockSpec((1, DH, NB // 2), lambda b, h: (h, 0, 0)),
        ],
        out_specs=pl.BlockSpec((N // 128, 128), lambda b, h: (b * H + h, 0)),
        out_shape=jax.ShapeDtypeStruct((M // 128, 128), jnp.int32),
    )(qkv_t, R)


def _sc_permute(qkv2d, idx2d):
    mesh = plsc.VectorSubcoreMesh(core_axis_name="c", subcore_axis_name="s")

    @functools.partial(
        pl.kernel, mesh=mesh,
        out_type=jax.ShapeDtypeStruct((M, 4 * DH), jnp.float32),
        scratch_types=[
            pltpu.VMEM((NJ, JROWS), jnp.int32),
            pltpu.VMEM((JROWS, 4 * DH), jnp.float32),
            pltpu.SemaphoreType.DMA,
        ],
    )
    def k(qkv_hbm, idx_hbm, out_hbm, idx_v, rows_v, sem):
        wid = lax.axis_index("s") * 2 + lax.axis_index("c")
        pltpu.sync_copy(idx_hbm.at[pl.ds(wid * NJ, NJ)], idx_v)

        def body(j, carry):
            base = wid * RPW + j * JROWS
            pltpu.sync_copy(qkv_hbm.at[pl.ds(base, JROWS)], rows_v)
            pltpu.async_copy(rows_v, out_hbm.at[idx_v.at[j]], sem).wait()
            return carry

        lax.fori_loop(0, NJ, body, 0)

    return k(qkv2d, idx2d)


def _sc_unsort(sout2d, idx2d):
    mesh = plsc.VectorSubcoreMesh(core_axis_name="c", subcore_axis_name="s")

    @functools.partial(
        pl.kernel, mesh=mesh,
        out_type=jax.ShapeDtypeStruct((M, 4 * DH), jnp.float32),
        scratch_types=[
            pltpu.VMEM((NJ, JROWS), jnp.int32),
            pltpu.VMEM((JROWS, 4 * DH), jnp.float32),
            pltpu.SemaphoreType.DMA,
        ],
    )
    def k(sout_hbm, idx_hbm, out_hbm, idx_v, rows_v, sem):
        wid = lax.axis_index("s") * 2 + lax.axis_index("c")
        pltpu.sync_copy(idx_hbm.at[pl.ds(wid * NJ, NJ)], idx_v)

        def body(j, carry):
            base = wid * RPW + j * JROWS
            pltpu.async_copy(sout_hbm.at[idx_v.at[j]], rows_v, sem).wait()
            pltpu.sync_copy(rows_v, out_hbm.at[pl.ds(base, JROWS)])
            return carry

        lax.fori_loop(0, NJ, body, 0)

    return k(sout2d, idx2d)


def _k3_body(sqkv_ref, out_ref):
    qk = sqkv_ref[0, :, :DH]
    v = sqkv_ref[0, :, DH:2 * DH]
    nrm = jnp.sqrt(jnp.sum(qk * qk, axis=1, keepdims=True))
    ck = qk / (nrm + 1e-6)
    scale = 1.0 / jnp.sqrt(jnp.float32(DH))
    for c in range(NB):
        cq = qk[c * CS:(c + 1) * CS, :]
        ckc = ck[c * CS:(c + 1) * CS, :]
        cv = v[c * CS:(c + 1) * CS, :]
        scores = lax.dot_general(cq, ckc, (((1,), (1,)), ((), ())),
                                 preferred_element_type=jnp.float32) * scale
        mx = jnp.max(scores, axis=1, keepdims=True)
        e = jnp.exp(scores - mx)
        attn = e / jnp.sum(e, axis=1, keepdims=True)
        o = jnp.dot(attn, cv, preferred_element_type=jnp.float32)
        out_ref[0, c * CS:(c + 1) * CS, :] = jnp.concatenate(
            [o, jnp.zeros((CS, 3 * DH), jnp.float32)], axis=1)


def _k3(sqkv_t):
    return pl.pallas_call(
        _k3_body,
        grid=(B * H,),
        in_specs=[pl.BlockSpec((1, N, 4 * DH), lambda i: (i, 0, 0))],
        out_specs=pl.BlockSpec((1, N, 4 * DH), lambda i: (i, 0, 0)),
        out_shape=jax.ShapeDtypeStruct((B * H, N, 4 * DH), jnp.float32),
    )(sqkv_t)


def _k4_body(ao_ref, x_ref, wo_ref, bo_ref, n2s_ref, n2b_ref,
             wa_ref, ba_ref, wb_ref, bb_ref, wc_ref, bc_ref, out_ref):
    o = jnp.dot(ao_ref[...], wo_ref[...], preferred_element_type=jnp.float32) + bo_ref[...]
    h = x_ref[...] + o
    h2 = _ln(h, n2s_ref[...], n2b_ref[...])
    a = jnp.dot(h2, wa_ref[...], preferred_element_type=jnp.float32) + ba_ref[...]
    g = jnp.dot(h2, wb_ref[...], preferred_element_type=jnp.float32) + bb_ref[...]
    g = a * jnp.maximum(g, 0.0)
    out_ref[...] = h + jnp.dot(g, wc_ref[...], preferred_element_type=jnp.float32) + bc_ref[...]


def _k4(ao2d, x2d, Wo, bo, n2s, n2b, Wa, ba, Wb, bb, Wc, bc):
    nsteps = (B * N) // ROWS_BLK
    row = pl.BlockSpec((ROWS_BLK, D), lambda i: (i, 0))
    return pl.pallas_call(
        _k4_body,
        grid=(nsteps,),
        in_specs=[
            row, row,
            pl.BlockSpec((D, D), lambda i: (0, 0)),
            pl.BlockSpec((1, D), lambda i: (0, 0)),
            pl.BlockSpec((1, D), lambda i: (0, 0)),
            pl.BlockSpec((1, D), lambda i: (0, 0)),
            pl.BlockSpec((D, F), lambda i: (0, 0)),
            pl.BlockSpec((1, F), lambda i: (0, 0)),
            pl.BlockSpec((D, F), lambda i: (0, 0)),
            pl.BlockSpec((1, F), lambda i: (0, 0)),
            pl.BlockSpec((F, D), lambda i: (0, 0)),
            pl.BlockSpec((1, D), lambda i: (0, 0)),
        ],
        out_specs=row,
        out_shape=jax.ShapeDtypeStruct((B * N, D), jnp.float32),
    )(ao2d, x2d, Wo, bo.reshape(1, D), n2s, n2b,
      Wa, ba.reshape(1, F), Wb, bb.reshape(1, F), Wc, bc.reshape(1, D))


def kernel(x, Wqk, bqk, Wv, bv, Wo, bo, R, n1_scale, n1_bias,
           Wa, ba, Wb, bb, Wc, bc, n2_scale, n2_bias):
    x2d = x.reshape(B * N, D)
    qk_r, v_r = _k1(x2d, Wqk, bqk, Wv, bv,
                    n1_scale.reshape(1, D), n1_bias.reshape(1, D))
    qk_t = qk_r.reshape(B, N, H, DH).transpose(0, 2, 1, 3)
    v_t = v_r.reshape(B, N, H, DH).transpose(0, 2, 1, 3)
    qkv_t = jnp.concatenate(
        [qk_t, v_t, jnp.zeros((B, H, N, 2 * DH), jnp.float32)],
        axis=-1).reshape(B * H, N, 4 * DH)

    idx2d = _k2(qkv_t, R)
    sqkv = _sc_permute(qkv_t.reshape(M, 4 * DH), idx2d)
    sout = _k3(sqkv.reshape(B * H, N, 4 * DH))
    out_t = _sc_unsort(sout.reshape(M, 4 * DH), idx2d)

    ao2d = out_t.reshape(B, H, N, 4 * DH)[..., :DH].transpose(
        0, 2, 1, 3).reshape(B * N, D)
    y = _k4(ao2d, x2d, Wo, bo, n2_scale.reshape(1, D), n2_bias.reshape(1, D),
            Wa, ba, Wb, bb, Wc, bc)
    return y.reshape(B, N, D)


# speedup vs baseline: 7.0291x; 1.7037x over previous
import functools

import jax
import jax.numpy as jnp
from jax import lax
from jax.experimental import pallas as pl
from jax.experimental.pallas import tpu as pltpu
from jax.experimental.pallas import tpu_sc as plsc

D = 1024
DH = 32
H = D // DH
NB = 32
B, N = 4, 2048
CS = N // NB
M = B * H * N
F = 3 * D

ROWS_BLK = 256
NW = 32
RPW = M // NW
JROWS = 128
NJ = RPW // JROWS


def _ln(xb, scale, bias):
    mu = jnp.mean(xb, axis=1, keepdims=True)
    xc = xb - mu
    var = jnp.sum(xc * xc, axis=1, keepdims=True) * (1.0 / (xb.shape[1] - 1))
    return xc * lax.rsqrt(var + 1e-6) * scale + bias


def _k1_body(x_ref, wqk_ref, bqk_ref, wv_ref, bv_ref, s_ref, b_ref,
             qk_ref, v_ref):
    h1 = _ln(x_ref[...], s_ref[...], b_ref[...])
    qk_ref[...] = jnp.dot(h1, wqk_ref[...], preferred_element_type=jnp.float32) + bqk_ref[...]
    v_ref[...] = jnp.dot(h1, wv_ref[...], preferred_element_type=jnp.float32) + bv_ref[...]


def _k1(x2d, Wqk, bqk, Wv, bv, n1s, n1b):
    nsteps = (B * N) // ROWS_BLK
    row_spec = pl.BlockSpec((ROWS_BLK, D), lambda i: (i, 0))
    full = pl.BlockSpec((D, D), lambda i: (0, 0))
    vec = pl.BlockSpec((1, D), lambda i: (0, 0))
    return pl.pallas_call(
        _k1_body,
        grid=(nsteps,),
        in_specs=[row_spec, full, vec, full, vec, vec, vec],
        out_specs=[row_spec, row_spec],
        out_shape=[jax.ShapeDtypeStruct((B * N, D), jnp.float32)] * 2,
    )(x2d, Wqk, bqk.reshape(1, D), Wv, bv.reshape(1, D), n1s, n1b)


def _k2_body(qk_ref, r_ref, idx_ref):
    qk = qk_ref[0]
    rot = jnp.dot(qk, r_ref[0], preferred_element_type=jnp.float32)
    conc = jnp.concatenate([rot, -rot], axis=1)
    mx = jnp.max(conc, axis=1, keepdims=True)
    lane = lax.broadcasted_iota(jnp.int32, (N, NB), 1).astype(jnp.float32)
    bucket = jnp.min(jnp.where(conc == mx, lane, float(NB)), axis=1,
                     keepdims=True)
    oh = (bucket == lane).astype(jnp.float32)

    gi = lax.broadcasted_iota(jnp.int32, (128, 128), 0)
    gj = lax.broadcasted_iota(jnp.int32, (128, 128), 1)
    t128 = (gj < gi).astype(jnp.float32)
    blocks = []
    prun = jnp.zeros((1, NB), jnp.float32)
    for g in range(N // 128):
        og = oh[g * 128:(g + 1) * 128, :]
        within = jnp.dot(t128, og, preferred_element_type=jnp.float32)
        blocks.append(within + prun)
        prun = prun + jnp.sum(og, axis=0, keepdims=True)
    counts = jnp.concatenate(blocks, axis=0)
    ui = lax.broadcasted_iota(jnp.int32, (NB, NB), 0)
    uj = lax.broadcasted_iota(jnp.int32, (NB, NB), 1)
    u32 = (ui < uj).astype(jnp.float32)
    prefix = jnp.dot(prun, u32, preferred_element_type=jnp.float32, precision=lax.Precision.HIGHEST)
    rank = jnp.sum((counts + prefix) * oh, axis=1, keepdims=True)

    ri = lax.broadcasted_iota(jnp.int32, (N, 128), 0)
    rj = lax.broadcasted_iota(jnp.int32, (N, 128), 1)
    spread = rank * (lax.rem(ri, 128) == rj).astype(jnp.float32)
    ai = lax.broadcasted_iota(jnp.int32, (N // 128, N), 0)
    aj = lax.broadcasted_iota(jnp.int32, (N // 128, N), 1)
    asel = (lax.div(aj, 128) == ai).astype(jnp.float32)
    bh = pl.program_id(0) * H + pl.program_id(1)
    local16 = jnp.dot(asel, spread, preferred_element_type=jnp.float32,
                      precision=lax.Precision.HIGHEST)
    idx_ref[...] = local16.astype(jnp.int32) + bh * N


def _k2(qk_t, R):
    return pl.pallas_call(
        _k2_body,
        grid=(B, H),
        in_specs=[
            pl.BlockSpec((1, N, DH), lambda b, h: (b * H + h, 0, 0)),
            pl.BlockSpec((1, DH, NB // 2), lambda b, h: (h, 0, 0)),
        ],
        out_specs=pl.BlockSpec((N // 128, 128), lambda b, h: (b * H + h, 0)),
        out_shape=jax.ShapeDtypeStruct((M // 128, 128), jnp.int32),
    )(qk_t, R)


def _sc_permute(qkv2d, idx2d):
    mesh = plsc.VectorSubcoreMesh(core_axis_name="c", subcore_axis_name="s")

    @functools.partial(
        pl.kernel, mesh=mesh,
        out_type=jax.ShapeDtypeStruct((M, 4 * DH), jnp.float32),
        scratch_types=[
            pltpu.VMEM((NJ, JROWS), jnp.int32),
            pltpu.VMEM((JROWS, 4 * DH), jnp.float32),
            pltpu.SemaphoreType.DMA,
        ],
    )
    def k(qkv_hbm, idx_hbm, out_hbm, idx_v, rows_v, sem):
        wid = lax.axis_index("s") * 2 + lax.axis_index("c")
        pltpu.sync_copy(idx_hbm.at[pl.ds(wid * NJ, NJ)], idx_v)

        def body(j, carry):
            base = wid * RPW + j * JROWS
            pltpu.sync_copy(qkv_hbm.at[pl.ds(base, JROWS)], rows_v)
            pltpu.async_copy(rows_v, out_hbm.at[idx_v.at[j]], sem).wait()
            return carry

        lax.fori_loop(0, NJ, body, 0)

    return k(qkv2d, idx2d)


def _sc_unsort(sout2d, idx2d):
    mesh = plsc.VectorSubcoreMesh(core_axis_name="c", subcore_axis_name="s")

    @functools.partial(
        pl.kernel, mesh=mesh,
        out_type=jax.ShapeDtypeStruct((M, 4 * DH), jnp.float32),
        scratch_types=[
            pltpu.VMEM((NJ, JROWS), jnp.int32),
            pltpu.VMEM((JROWS, 4 * DH), jnp.float32),
            pltpu.SemaphoreType.DMA,
        ],
    )
    def k(sout_hbm, idx_hbm, out_hbm, idx_v, rows_v, sem):
        wid = lax.axis_index("s") * 2 + lax.axis_index("c")
        pltpu.sync_copy(idx_hbm.at[pl.ds(wid * NJ, NJ)], idx_v)

        def body(j, carry):
            base = wid * RPW + j * JROWS
            pltpu.async_copy(sout_hbm.at[idx_v.at[j]], rows_v, sem).wait()
            pltpu.sync_copy(rows_v, out_hbm.at[pl.ds(base, JROWS)])
            return carry

        lax.fori_loop(0, NJ, body, 0)

    return k(sout2d, idx2d)


BAND = 4 * CS


def _k3_body(sqkv_ref, out_ref):
    qk32 = sqkv_ref[0, :, :DH]
    v = sqkv_ref[0, :, DH:2 * DH].astype(jnp.bfloat16)
    qk = qk32.astype(jnp.bfloat16)
    nrm = jnp.sqrt(jnp.sum(qk32 * qk32, axis=1, keepdims=True))
    ck = (qk32 / (nrm + 1e-6)).astype(jnp.bfloat16)
    scale = 1.0 / jnp.sqrt(jnp.float32(DH))
    bi = lax.broadcasted_iota(jnp.int32, (BAND, BAND), 0)
    bj = lax.broadcasted_iota(jnp.int32, (BAND, BAND), 1)
    offblock = lax.div(bi, CS) != lax.div(bj, CS)
    for b0 in range(N // BAND):
        sl = slice(b0 * BAND, (b0 + 1) * BAND)
        scores = lax.dot_general(qk[sl, :], ck[sl, :], (((1,), (1,)), ((), ())),
                                 preferred_element_type=jnp.float32) * scale
        scores = jnp.where(offblock, -1e30, scores)
        mx = jnp.max(scores, axis=1, keepdims=True)
        e = jnp.exp(scores - mx)
        attn = (e / jnp.sum(e, axis=1, keepdims=True)).astype(jnp.bfloat16)
        o = jnp.dot(attn, v[sl, :], preferred_element_type=jnp.float32)
        out_ref[0, sl, :] = jnp.concatenate(
            [o, jnp.zeros((BAND, 3 * DH), jnp.float32)], axis=1)


def _k3(sqkv_t):
    return pl.pallas_call(
        _k3_body,
        grid=(B * H,),
        in_specs=[pl.BlockSpec((1, N, 4 * DH), lambda i: (i, 0, 0))],
        out_specs=pl.BlockSpec((1, N, 4 * DH), lambda i: (i, 0, 0)),
        out_shape=jax.ShapeDtypeStruct((B * H, N, 4 * DH), jnp.float32),
    )(sqkv_t)


def _k4_body(ao_ref, x_ref, wo_ref, bo_ref, n2s_ref, n2b_ref,
             wa_ref, ba_ref, wb_ref, bb_ref, wc_ref, bc_ref, out_ref):
    o = jnp.dot(ao_ref[...].astype(jnp.bfloat16), wo_ref[...],
                preferred_element_type=jnp.float32) + bo_ref[...]
    h = x_ref[...] + o
    h2 = _ln(h, n2s_ref[...], n2b_ref[...]).astype(jnp.bfloat16)
    a = jnp.dot(h2, wa_ref[...], preferred_element_type=jnp.float32) + ba_ref[...]
    g = jnp.dot(h2, wb_ref[...], preferred_element_type=jnp.float32) + bb_ref[...]
    g = (a * jnp.maximum(g, 0.0)).astype(jnp.bfloat16)
    out_ref[...] = h + jnp.dot(g, wc_ref[...], preferred_element_type=jnp.float32) + bc_ref[...]


def _k4(ao2d, x2d, Wo, bo, n2s, n2b, Wa, ba, Wb, bb, Wc, bc):
    nsteps = (B * N) // ROWS_BLK
    row = pl.BlockSpec((ROWS_BLK, D), lambda i: (i, 0))
    return pl.pallas_call(
        _k4_body,
        grid=(nsteps,),
        in_specs=[
            row, row,
            pl.BlockSpec((D, D), lambda i: (0, 0)),
            pl.BlockSpec((1, D), lambda i: (0, 0)),
            pl.BlockSpec((1, D), lambda i: (0, 0)),
            pl.BlockSpec((1, D), lambda i: (0, 0)),
            pl.BlockSpec((D, F), lambda i: (0, 0)),
            pl.BlockSpec((1, F), lambda i: (0, 0)),
            pl.BlockSpec((D, F), lambda i: (0, 0)),
            pl.BlockSpec((1, F), lambda i: (0, 0)),
            pl.BlockSpec((F, D), lambda i: (0, 0)),
            pl.BlockSpec((1, D), lambda i: (0, 0)),
        ],
        out_specs=row,
        out_shape=jax.ShapeDtypeStruct((B * N, D), jnp.float32),
    )(ao2d, x2d, Wo.astype(jnp.bfloat16), bo.reshape(1, D), n2s, n2b,
      Wa.astype(jnp.bfloat16), ba.reshape(1, F),
      Wb.astype(jnp.bfloat16), bb.reshape(1, F),
      Wc.astype(jnp.bfloat16), bc.reshape(1, D))


def kernel(x, Wqk, bqk, Wv, bv, Wo, bo, R, n1_scale, n1_bias,
           Wa, ba, Wb, bb, Wc, bc, n2_scale, n2_bias):
    x2d = x.reshape(B * N, D)
    qk_r, v_r = _k1(x2d, Wqk, bqk, Wv, bv,
                    n1_scale.reshape(1, D), n1_bias.reshape(1, D))
    qk_t = qk_r.reshape(B, N, H, DH).transpose(0, 2, 1, 3)
    v_t = v_r.reshape(B, N, H, DH).transpose(0, 2, 1, 3)
    qkv_t = jnp.concatenate(
        [qk_t, v_t, jnp.zeros((B, H, N, 2 * DH), jnp.float32)],
        axis=-1).reshape(B * H, N, 4 * DH)

    idx2d = _k2(qk_t.reshape(B * H, N, DH), R)
    sqkv = _sc_permute(qkv_t.reshape(M, 4 * DH), idx2d)
    sout = _k3(sqkv.reshape(B * H, N, 4 * DH))
    out_t = _sc_unsort(sout.reshape(M, 4 * DH), idx2d)

    ao2d = out_t.reshape(B, H, N, 4 * DH)[..., :DH].transpose(
        0, 2, 1, 3).reshape(B * N, D)
    y = _k4(ao2d, x2d, Wo, bo, n2_scale.reshape(1, D), n2_bias.reshape(1, D),
            Wa, ba, Wb, bb, Wc, bc)
    return y.reshape(B, N, D)


# speedup vs baseline: 7.9359x; 1.1290x over previous
import functools

import jax
import jax.numpy as jnp
from jax import lax
from jax.experimental import pallas as pl
from jax.experimental.pallas import tpu as pltpu
from jax.experimental.pallas import tpu_sc as plsc

D = 1024
DH = 32
H = D // DH
NB = 32
B, N = 4, 2048
CS = N // NB
M = B * H * N
F = 3 * D

ROWS_BLK = 256
NW = 32
RPW = M // NW
JROWS = 128
NJ = RPW // JROWS
DEPTH = 4


def _ln(xb, scale, bias):
    mu = jnp.mean(xb, axis=1, keepdims=True)
    xc = xb - mu
    var = jnp.sum(xc * xc, axis=1, keepdims=True) * (1.0 / (xb.shape[1] - 1))
    return xc * lax.rsqrt(var + 1e-6) * scale + bias


def _k1_body(x_ref, wqk_ref, bqk_ref, wv_ref, bv_ref, s_ref, b_ref,
             qk_ref, v_ref):
    h1 = _ln(x_ref[...], s_ref[...], b_ref[...])
    qk_ref[...] = jnp.dot(h1, wqk_ref[...], preferred_element_type=jnp.float32) + bqk_ref[...]
    v_ref[...] = jnp.dot(h1, wv_ref[...], preferred_element_type=jnp.float32) + bv_ref[...]


def _k1(x2d, Wqk, bqk, Wv, bv, n1s, n1b):
    nsteps = (B * N) // ROWS_BLK
    row_spec = pl.BlockSpec((ROWS_BLK, D), lambda i: (i, 0))
    full = pl.BlockSpec((D, D), lambda i: (0, 0))
    vec = pl.BlockSpec((1, D), lambda i: (0, 0))
    return pl.pallas_call(
        _k1_body,
        grid=(nsteps,),
        in_specs=[row_spec, full, vec, full, vec, vec, vec],
        out_specs=[row_spec, row_spec],
        out_shape=[jax.ShapeDtypeStruct((B * N, D), jnp.float32)] * 2,
    )(x2d, Wqk, bqk.reshape(1, D), Wv, bv.reshape(1, D), n1s, n1b)


def _k2_body(qk_ref, r_ref, idx_ref):
    qk = qk_ref[0]
    rot = jnp.dot(qk, r_ref[0], preferred_element_type=jnp.float32)
    lane16 = lax.broadcasted_iota(jnp.int32, (N, NB // 2), 1).astype(jnp.float32)
    mxp = jnp.max(rot, axis=1, keepdims=True)
    mxn = jnp.min(rot, axis=1, keepdims=True)
    ip = jnp.min(jnp.where(rot == mxp, lane16, float(NB)), axis=1, keepdims=True)
    iq = jnp.min(jnp.where(rot == mxn, lane16, float(NB)), axis=1, keepdims=True)
    bucket = jnp.where(mxp >= -mxn, ip, iq + float(NB // 2))
    lane = lax.broadcasted_iota(jnp.int32, (N, NB), 1).astype(jnp.float32)
    oh = (bucket == lane).astype(jnp.float32)

    gi = lax.broadcasted_iota(jnp.int32, (128, 128), 0)
    gj = lax.broadcasted_iota(jnp.int32, (128, 128), 1)
    t128 = (gj < gi).astype(jnp.float32)
    blocks = []
    prun = jnp.zeros((1, NB), jnp.float32)
    for g in range(N // 128):
        og = oh[g * 128:(g + 1) * 128, :]
        within = jnp.dot(t128, og, preferred_element_type=jnp.float32)
        blocks.append(within + prun)
        prun = prun + jnp.sum(og, axis=0, keepdims=True)
    ui = lax.broadcasted_iota(jnp.int32, (NB, NB), 0)
    uj = lax.broadcasted_iota(jnp.int32, (NB, NB), 1)
    u32 = (ui < uj).astype(jnp.float32)
    prefix = jnp.dot(prun, u32, preferred_element_type=jnp.float32, precision=lax.Precision.HIGHEST)
    cols = []
    for g in range(N // 128):
        ohg = oh[g * 128:(g + 1) * 128, :]
        cols.append(jnp.sum((blocks[g] + prefix) * ohg, axis=1, keepdims=True))
    rankmat = jnp.concatenate(cols, axis=1)
    bh = pl.program_id(0) * H + pl.program_id(1)
    idx_ref[...] = jnp.transpose(rankmat).astype(jnp.int32) + bh * N


def _k2(qk_t, R):
    return pl.pallas_call(
        _k2_body,
        grid=(B, H),
        in_specs=[
            pl.BlockSpec((1, N, DH), lambda b, h: (b * H + h, 0, 0)),
            pl.BlockSpec((1, DH, NB // 2), lambda b, h: (h, 0, 0)),
        ],
        out_specs=pl.BlockSpec((N // 128, 128), lambda b, h: (b * H + h, 0)),
        out_shape=jax.ShapeDtypeStruct((M // 128, 128), jnp.int32),
    )(qk_t, R)


def _sc_permute(qkv2d, idx2d):
    mesh = plsc.VectorSubcoreMesh(core_axis_name="c", subcore_axis_name="s")

    @functools.partial(
        pl.kernel, mesh=mesh,
        out_type=jax.ShapeDtypeStruct((M, 4 * DH), jnp.float32),
        scratch_types=[
            pltpu.VMEM((NJ, JROWS), jnp.int32),
            pltpu.VMEM((DEPTH, JROWS, 4 * DH), jnp.float32),
            pltpu.SemaphoreType.DMA,
            pltpu.SemaphoreType.DMA,
        ],
    )
    def k(qkv_hbm, idx_hbm, out_hbm, idx_v, rows_v, lsem, ssem):
        wid = lax.axis_index("s") * 2 + lax.axis_index("c")
        pltpu.sync_copy(idx_hbm.at[pl.ds(wid * NJ, NJ)], idx_v)

        def body(q, carry):
            j0 = q * DEPTH
            loads = []
            for b in range(DEPTH):
                base = wid * RPW + (j0 + b) * JROWS
                loads.append(pltpu.async_copy(
                    qkv_hbm.at[pl.ds(base, JROWS)], rows_v.at[b], lsem))
            for h in loads:
                h.wait()
            stores = []
            for b in range(DEPTH):
                stores.append(pltpu.async_copy(
                    rows_v.at[b], out_hbm.at[idx_v.at[j0 + b]], ssem))
            for h in stores:
                h.wait()
            return carry

        lax.fori_loop(0, NJ // DEPTH, body, 0)

    return k(qkv2d, idx2d)


def _sc_unsort(sout2d, idx2d):
    mesh = plsc.VectorSubcoreMesh(core_axis_name="c", subcore_axis_name="s")

    @functools.partial(
        pl.kernel, mesh=mesh,
        out_type=jax.ShapeDtypeStruct((M, 4 * DH), jnp.float32),
        scratch_types=[
            pltpu.VMEM((NJ, JROWS), jnp.int32),
            pltpu.VMEM((DEPTH, JROWS, 4 * DH), jnp.float32),
            pltpu.SemaphoreType.DMA,
            pltpu.SemaphoreType.DMA,
        ],
    )
    def k(sout_hbm, idx_hbm, out_hbm, idx_v, rows_v, gsem, wsem):
        wid = lax.axis_index("s") * 2 + lax.axis_index("c")
        pltpu.sync_copy(idx_hbm.at[pl.ds(wid * NJ, NJ)], idx_v)

        def body(q, carry):
            j0 = q * DEPTH
            gathers = []
            for b in range(DEPTH):
                gathers.append(pltpu.async_copy(
                    sout_hbm.at[idx_v.at[j0 + b]], rows_v.at[b], gsem))
            for h in gathers:
                h.wait()
            writes = []
            for b in range(DEPTH):
                base = wid * RPW + (j0 + b) * JROWS
                writes.append(pltpu.async_copy(
                    rows_v.at[b], out_hbm.at[pl.ds(base, JROWS)], wsem))
            for h in writes:
                h.wait()
            return carry

        lax.fori_loop(0, NJ // DEPTH, body, 0)

    return k(sout2d, idx2d)


BAND = 4 * CS


def _k3_body(sqkv_ref, out_ref):
    qk32 = sqkv_ref[0, :, :DH]
    v = sqkv_ref[0, :, DH:2 * DH].astype(jnp.bfloat16)
    qk = qk32.astype(jnp.bfloat16)
    nrm = jnp.sqrt(jnp.sum(qk32 * qk32, axis=1, keepdims=True))
    ck = (qk32 / (nrm + 1e-6)).astype(jnp.bfloat16)
    scale = 1.0 / jnp.sqrt(jnp.float32(DH))
    bi = lax.broadcasted_iota(jnp.int32, (BAND, BAND), 0)
    bj = lax.broadcasted_iota(jnp.int32, (BAND, BAND), 1)
    offblock = lax.div(bi, CS) != lax.div(bj, CS)
    for b0 in range(N // BAND):
        sl = slice(b0 * BAND, (b0 + 1) * BAND)
        scores = lax.dot_general(qk[sl, :], ck[sl, :], (((1,), (1,)), ((), ())),
                                 preferred_element_type=jnp.float32) * scale
        e = jnp.exp(jnp.where(offblock, -1e30, scores))
        attn = (e * (1.0 / jnp.sum(e, axis=1, keepdims=True))).astype(jnp.bfloat16)
        o = jnp.dot(attn, v[sl, :], preferred_element_type=jnp.float32)
        out_ref[0, sl, :DH] = o


def _k3(sqkv_t):
    return pl.pallas_call(
        _k3_body,
        grid=(B * H,),
        in_specs=[pl.BlockSpec((1, N, 4 * DH), lambda i: (i, 0, 0))],
        out_specs=pl.BlockSpec((1, N, 4 * DH), lambda i: (i, 0, 0)),
        out_shape=jax.ShapeDtypeStruct((B * H, N, 4 * DH), jnp.float32),
    )(sqkv_t)


def _k4_body(ao_ref, x_ref, wo_ref, bo_ref, n2s_ref, n2b_ref,
             wa_ref, ba_ref, wb_ref, bb_ref, wc_ref, bc_ref, out_ref):
    o = jnp.dot(ao_ref[...].astype(jnp.bfloat16), wo_ref[...],
                preferred_element_type=jnp.float32) + bo_ref[...]
    h = x_ref[...] + o
    h2 = _ln(h, n2s_ref[...], n2b_ref[...]).astype(jnp.bfloat16)
    a = jnp.dot(h2, wa_ref[...], preferred_element_type=jnp.float32) + ba_ref[...]
    g = jnp.dot(h2, wb_ref[...], preferred_element_type=jnp.float32) + bb_ref[...]
    g = (a * jnp.maximum(g, 0.0)).astype(jnp.bfloat16)
    out_ref[...] = h + jnp.dot(g, wc_ref[...], preferred_element_type=jnp.float32) + bc_ref[...]


def _k4(ao2d, x2d, Wo, bo, n2s, n2b, Wa, ba, Wb, bb, Wc, bc):
    nsteps = (B * N) // ROWS_BLK
    row = pl.BlockSpec((ROWS_BLK, D), lambda i: (i, 0))
    return pl.pallas_call(
        _k4_body,
        grid=(nsteps,),
        in_specs=[
            row, row,
            pl.BlockSpec((D, D), lambda i: (0, 0)),
            pl.BlockSpec((1, D), lambda i: (0, 0)),
            pl.BlockSpec((1, D), lambda i: (0, 0)),
            pl.BlockSpec((1, D), lambda i: (0, 0)),
            pl.BlockSpec((D, F), lambda i: (0, 0)),
            pl.BlockSpec((1, F), lambda i: (0, 0)),
            pl.BlockSpec((D, F), lambda i: (0, 0)),
            pl.BlockSpec((1, F), lambda i: (0, 0)),
            pl.BlockSpec((F, D), lambda i: (0, 0)),
            pl.BlockSpec((1, D), lambda i: (0, 0)),
        ],
        out_specs=row,
        out_shape=jax.ShapeDtypeStruct((B * N, D), jnp.float32),
    )(ao2d, x2d, Wo.astype(jnp.bfloat16), bo.reshape(1, D), n2s, n2b,
      Wa.astype(jnp.bfloat16), ba.reshape(1, F),
      Wb.astype(jnp.bfloat16), bb.reshape(1, F),
      Wc.astype(jnp.bfloat16), bc.reshape(1, D))


def kernel(x, Wqk, bqk, Wv, bv, Wo, bo, R, n1_scale, n1_bias,
           Wa, ba, Wb, bb, Wc, bc, n2_scale, n2_bias):
    x2d = x.reshape(B * N, D)
    qk_r, v_r = _k1(x2d, Wqk, bqk, Wv, bv,
                    n1_scale.reshape(1, D), n1_bias.reshape(1, D))
    qk_t = qk_r.reshape(B, N, H, DH).transpose(0, 2, 1, 3)
    v_t = v_r.reshape(B, N, H, DH).transpose(0, 2, 1, 3)
    qkv_t = jnp.concatenate(
        [qk_t, v_t, jnp.zeros((B, H, N, 2 * DH), jnp.float32)],
        axis=-1).reshape(B * H, N, 4 * DH)

    idx2d = _k2(qk_t.reshape(B * H, N, DH), R)
    sqkv = _sc_permute(qkv_t.reshape(M, 4 * DH), idx2d)
    sout = _k3(sqkv.reshape(B * H, N, 4 * DH))
    out_t = _sc_unsort(sout.reshape(M, 4 * DH), idx2d)

    ao2d = out_t.reshape(B, H, N, 4 * DH)[..., :DH].transpose(
        0, 2, 1, 3).reshape(B * N, D)
    y = _k4(ao2d, x2d, Wo, bo, n2_scale.reshape(1, D), n2_bias.reshape(1, D),
            Wa, ba, Wb, bb, Wc, bc)
    return y.reshape(B, N, D)


# speedup vs baseline: 9.7704x; 1.2312x over previous
import functools

import jax
import jax.numpy as jnp
from jax import lax
from jax.experimental import pallas as pl
from jax.experimental.pallas import tpu as pltpu
from jax.experimental.pallas import tpu_sc as plsc

D = 1024
DH = 32
H = D // DH
NB = 32
B, N = 4, 2048
CS = N // NB
M = B * H * N
F = 3 * D

ROWS_BLK = 256
NW = 32
RPW = M // NW
JROWS = 128
NJ = RPW // JROWS
DEPTH = 4


def _ln(xb, scale, bias):
    mu = jnp.mean(xb, axis=1, keepdims=True)
    xc = xb - mu
    var = jnp.sum(xc * xc, axis=1, keepdims=True) * (1.0 / (xb.shape[1] - 1))
    return xc * lax.rsqrt(var + 1e-6) * scale + bias


def _k1_body(x_ref, wqk_ref, bqk_ref, wv_ref, bv_ref, s_ref, b_ref, out_ref):
    h1 = _ln(x_ref[0], s_ref[...], b_ref[...])
    qk = jnp.dot(h1, wqk_ref[...], preferred_element_type=jnp.float32) + bqk_ref[...]
    v = jnp.dot(h1, wv_ref[...], preferred_element_type=jnp.float32) + bv_ref[...]
    for h in range(H):
        out_ref[h, :, :2 * DH] = jnp.concatenate(
            [qk[:, h * DH:(h + 1) * DH], v[:, h * DH:(h + 1) * DH]], axis=1)


def _k1(x, Wqk, bqk, Wv, bv, n1s, n1b):
    nb = N // ROWS_BLK
    full = pl.BlockSpec((D, D), lambda b, i: (0, 0))
    vec = pl.BlockSpec((1, D), lambda b, i: (0, 0))
    return pl.pallas_call(
        _k1_body,
        grid=(B, nb),
        in_specs=[pl.BlockSpec((1, ROWS_BLK, D), lambda b, i: (b, i, 0)),
                  full, vec, full, vec, vec, vec],
        out_specs=pl.BlockSpec((H, ROWS_BLK, 4 * DH), lambda b, i: (b, i, 0)),
        out_shape=jax.ShapeDtypeStruct((B * H, N, 4 * DH), jnp.float32),
    )(x, Wqk, bqk.reshape(1, D), Wv, bv.reshape(1, D), n1s, n1b)


def _k2_body(qkv_ref, r_ref, idx_ref):
    qk = qkv_ref[0, :, :DH]
    rot = jnp.dot(qk, r_ref[0], preferred_element_type=jnp.float32)
    lane16 = lax.broadcasted_iota(jnp.int32, (N, NB // 2), 1).astype(jnp.float32)
    mxp = jnp.max(rot, axis=1, keepdims=True)
    mxn = jnp.min(rot, axis=1, keepdims=True)
    ip = jnp.min(jnp.where(rot == mxp, lane16, float(NB)), axis=1, keepdims=True)
    iq = jnp.min(jnp.where(rot == mxn, lane16, float(NB)), axis=1, keepdims=True)
    bucket = jnp.where(mxp >= -mxn, ip, iq + float(NB // 2))
    lane = lax.broadcasted_iota(jnp.int32, (N, NB), 1).astype(jnp.float32)
    oh = (bucket == lane).astype(jnp.float32)

    gi = lax.broadcasted_iota(jnp.int32, (128, 128), 0)
    gj = lax.broadcasted_iota(jnp.int32, (128, 128), 1)
    t128 = (gj < gi).astype(jnp.float32)
    blocks = []
    prun = jnp.zeros((1, NB), jnp.float32)
    for g in range(N // 128):
        og = oh[g * 128:(g + 1) * 128, :]
        within = jnp.dot(t128, og, preferred_element_type=jnp.float32)
        blocks.append(within + prun)
        prun = prun + jnp.sum(og, axis=0, keepdims=True)
    ui = lax.broadcasted_iota(jnp.int32, (NB, NB), 0)
    uj = lax.broadcasted_iota(jnp.int32, (NB, NB), 1)
    u32 = (ui < uj).astype(jnp.float32)
    prefix = jnp.dot(prun, u32, preferred_element_type=jnp.float32, precision=lax.Precision.HIGHEST)
    cols = []
    for g in range(N // 128):
        ohg = oh[g * 128:(g + 1) * 128, :]
        cols.append(jnp.sum((blocks[g] + prefix) * ohg, axis=1, keepdims=True))
    rankmat = jnp.concatenate(cols, axis=1)
    bh = pl.program_id(0) * H + pl.program_id(1)
    idx_ref[...] = jnp.transpose(rankmat).astype(jnp.int32) + bh * N


def _k2(qk_t, R):
    return pl.pallas_call(
        _k2_body,
        grid=(B, H),
        in_specs=[
            pl.BlockSpec((1, N, 4 * DH), lambda b, h: (b * H + h, 0, 0)),
            pl.BlockSpec((1, DH, NB // 2), lambda b, h: (h, 0, 0)),
        ],
        out_specs=pl.BlockSpec((N // 128, 128), lambda b, h: (b * H + h, 0)),
        out_shape=jax.ShapeDtypeStruct((M // 128, 128), jnp.int32),
    )(qk_t, R)


def _sc_permute(qkv2d, idx2d):
    mesh = plsc.VectorSubcoreMesh(core_axis_name="c", subcore_axis_name="s")

    @functools.partial(
        pl.kernel, mesh=mesh,
        out_type=jax.ShapeDtypeStruct((M, 4 * DH), jnp.float32),
        scratch_types=[
            pltpu.VMEM((NJ, JROWS), jnp.int32),
            pltpu.VMEM((DEPTH, JROWS, 4 * DH), jnp.float32),
            pltpu.SemaphoreType.DMA,
            pltpu.SemaphoreType.DMA,
        ],
    )
    def k(qkv_hbm, idx_hbm, out_hbm, idx_v, rows_v, lsem, ssem):
        wid = lax.axis_index("s") * 2 + lax.axis_index("c")
        pltpu.sync_copy(idx_hbm.at[pl.ds(wid * NJ, NJ)], idx_v)

        def body(q, carry):
            j0 = q * DEPTH
            loads = []
            for b in range(DEPTH):
                base = wid * RPW + (j0 + b) * JROWS
                loads.append(pltpu.async_copy(
                    qkv_hbm.at[pl.ds(base, JROWS)], rows_v.at[b], lsem))
            for h in loads:
                h.wait()
            stores = []
            for b in range(DEPTH):
                stores.append(pltpu.async_copy(
                    rows_v.at[b], out_hbm.at[idx_v.at[j0 + b]], ssem))
            for h in stores:
                h.wait()
            return carry

        lax.fori_loop(0, NJ // DEPTH, body, 0)

    return k(qkv2d, idx2d)


def _sc_unsort(sout2d, idx2d):
    mesh = plsc.VectorSubcoreMesh(core_axis_name="c", subcore_axis_name="s")

    @functools.partial(
        pl.kernel, mesh=mesh,
        out_type=jax.ShapeDtypeStruct((M, 4 * DH), jnp.float32),
        scratch_types=[
            pltpu.VMEM((NJ, JROWS), jnp.int32),
            pltpu.VMEM((DEPTH, JROWS, 4 * DH), jnp.float32),
            pltpu.SemaphoreType.DMA,
            pltpu.SemaphoreType.DMA,
        ],
    )
    def k(sout_hbm, idx_hbm, out_hbm, idx_v, rows_v, gsem, wsem):
        wid = lax.axis_index("s") * 2 + lax.axis_index("c")
        pltpu.sync_copy(idx_hbm.at[pl.ds(wid * NJ, NJ)], idx_v)

        def body(q, carry):
            j0 = q * DEPTH
            gathers = []
            for b in range(DEPTH):
                gathers.append(pltpu.async_copy(
                    sout_hbm.at[idx_v.at[j0 + b]], rows_v.at[b], gsem))
            for h in gathers:
                h.wait()
            writes = []
            for b in range(DEPTH):
                base = wid * RPW + (j0 + b) * JROWS
                writes.append(pltpu.async_copy(
                    rows_v.at[b], out_hbm.at[pl.ds(base, JROWS)], wsem))
            for h in writes:
                h.wait()
            return carry

        lax.fori_loop(0, NJ // DEPTH, body, 0)

    return k(sout2d, idx2d)


BAND = 4 * CS


def _k3_body(sqkv_ref, out_ref):
    qk32 = sqkv_ref[0, :, :DH]
    v = sqkv_ref[0, :, DH:2 * DH].astype(jnp.bfloat16)
    qk = qk32.astype(jnp.bfloat16)
    nrm = jnp.sqrt(jnp.sum(qk32 * qk32, axis=1, keepdims=True))
    ck = (qk32 / (nrm + 1e-6)).astype(jnp.bfloat16)
    scale = 1.0 / jnp.sqrt(jnp.float32(DH))
    bi = lax.broadcasted_iota(jnp.int32, (BAND, BAND), 0)
    bj = lax.broadcasted_iota(jnp.int32, (BAND, BAND), 1)
    offblock = lax.div(bi, CS) != lax.div(bj, CS)
    for b0 in range(N // BAND):
        sl = slice(b0 * BAND, (b0 + 1) * BAND)
        scores = lax.dot_general(qk[sl, :], ck[sl, :], (((1,), (1,)), ((), ())),
                                 preferred_element_type=jnp.float32) * scale
        e = jnp.exp(jnp.where(offblock, -1e30, scores))
        attn = (e * (1.0 / jnp.sum(e, axis=1, keepdims=True))).astype(jnp.bfloat16)
        o = jnp.dot(attn, v[sl, :], preferred_element_type=jnp.float32)
        out_ref[0, sl, :DH] = o


def _k3(sqkv_t):
    return pl.pallas_call(
        _k3_body,
        grid=(B * H,),
        in_specs=[pl.BlockSpec((1, N, 4 * DH), lambda i: (i, 0, 0))],
        out_specs=pl.BlockSpec((1, N, 4 * DH), lambda i: (i, 0, 0)),
        out_shape=jax.ShapeDtypeStruct((B * H, N, 4 * DH), jnp.float32),
    )(sqkv_t)


def _k4_body(ao_ref, x_ref, wo_ref, bo_ref, n2s_ref, n2b_ref,
             wa_ref, ba_ref, wb_ref, bb_ref, wc_ref, bc_ref, out_ref):
    o = jnp.dot(ao_ref[...].astype(jnp.bfloat16), wo_ref[...],
                preferred_element_type=jnp.float32) + bo_ref[...]
    h = x_ref[...] + o
    h2 = _ln(h, n2s_ref[...], n2b_ref[...]).astype(jnp.bfloat16)
    a = jnp.dot(h2, wa_ref[...], preferred_element_type=jnp.float32) + ba_ref[...]
    g = jnp.dot(h2, wb_ref[...], preferred_element_type=jnp.float32) + bb_ref[...]
    g = (a * jnp.maximum(g, 0.0)).astype(jnp.bfloat16)
    out_ref[...] = h + jnp.dot(g, wc_ref[...], preferred_element_type=jnp.float32) + bc_ref[...]


def _k4(ao2d, x2d, Wo, bo, n2s, n2b, Wa, ba, Wb, bb, Wc, bc):
    nsteps = (B * N) // ROWS_BLK
    row = pl.BlockSpec((ROWS_BLK, D), lambda i: (i, 0))
    return pl.pallas_call(
        _k4_body,
        grid=(nsteps,),
        in_specs=[
            row, row,
            pl.BlockSpec((D, D), lambda i: (0, 0)),
            pl.BlockSpec((1, D), lambda i: (0, 0)),
            pl.BlockSpec((1, D), lambda i: (0, 0)),
            pl.BlockSpec((1, D), lambda i: (0, 0)),
            pl.BlockSpec((D, F), lambda i: (0, 0)),
            pl.BlockSpec((1, F), lambda i: (0, 0)),
            pl.BlockSpec((D, F), lambda i: (0, 0)),
            pl.BlockSpec((1, F), lambda i: (0, 0)),
            pl.BlockSpec((F, D), lambda i: (0, 0)),
            pl.BlockSpec((1, D), lambda i: (0, 0)),
        ],
        out_specs=row,
        out_shape=jax.ShapeDtypeStruct((B * N, D), jnp.float32),
    )(ao2d, x2d, Wo.astype(jnp.bfloat16), bo.reshape(1, D), n2s, n2b,
      Wa.astype(jnp.bfloat16), ba.reshape(1, F),
      Wb.astype(jnp.bfloat16), bb.reshape(1, F),
      Wc.astype(jnp.bfloat16), bc.reshape(1, D))


def kernel(x, Wqk, bqk, Wv, bv, Wo, bo, R, n1_scale, n1_bias,
           Wa, ba, Wb, bb, Wc, bc, n2_scale, n2_bias):
    x2d = x.reshape(B * N, D)
    qkv_t = _k1(x, Wqk, bqk, Wv, bv,
                n1_scale.reshape(1, D), n1_bias.reshape(1, D))

    idx2d = _k2(qkv_t, R)
    sqkv = _sc_permute(qkv_t.reshape(M, 4 * DH), idx2d)
    sout = _k3(sqkv.reshape(B * H, N, 4 * DH))
    out_t = _sc_unsort(sout.reshape(M, 4 * DH), idx2d)

    ao2d = out_t.reshape(B, H, N, 4 * DH)[..., :DH].transpose(
        0, 2, 1, 3).reshape(B * N, D)
    y = _k4(ao2d, x2d, Wo, bo, n2_scale.reshape(1, D), n2_bias.reshape(1, D),
            Wa, ba, Wb, bb, Wc, bc)
    return y.reshape(B, N, D)


# speedup vs baseline: 10.6256x; 1.0875x over previous
import functools

import jax
import jax.numpy as jnp
from jax import lax
from jax.experimental import pallas as pl
from jax.experimental.pallas import tpu as pltpu
from jax.experimental.pallas import tpu_sc as plsc

D = 1024
DH = 32
H = D // DH
NB = 32
B, N = 4, 2048
CS = N // NB
M = B * H * N
F = 3 * D

ROWS_BLK = 256
NW = 32
RPW = M // NW
JROWS = 128
NJ = RPW // JROWS
SET = 2


def _ln(xb, scale, bias):
    mu = jnp.mean(xb, axis=1, keepdims=True)
    xc = xb - mu
    var = jnp.sum(xc * xc, axis=1, keepdims=True) * (1.0 / (xb.shape[1] - 1))
    return xc * lax.rsqrt(var + 1e-6) * scale + bias


def _k1_body(x_ref, wqk_ref, bqk_ref, wv_ref, bv_ref, s_ref, b_ref, out_ref):
    h1 = _ln(x_ref[0], s_ref[...], b_ref[...])
    qk = jnp.dot(h1, wqk_ref[...], preferred_element_type=jnp.float32) + bqk_ref[...]
    v = jnp.dot(h1, wv_ref[...], preferred_element_type=jnp.float32) + bv_ref[...]
    for h in range(H):
        out_ref[h, :, :2 * DH] = jnp.concatenate(
            [qk[:, h * DH:(h + 1) * DH], v[:, h * DH:(h + 1) * DH]], axis=1)


def _k1(x, Wqk, bqk, Wv, bv, n1s, n1b):
    nb = N // ROWS_BLK
    full = pl.BlockSpec((D, D), lambda b, i: (0, 0))
    vec = pl.BlockSpec((1, D), lambda b, i: (0, 0))
    return pl.pallas_call(
        _k1_body,
        grid=(B, nb),
        in_specs=[pl.BlockSpec((1, ROWS_BLK, D), lambda b, i: (b, i, 0)),
                  full, vec, full, vec, vec, vec],
        out_specs=pl.BlockSpec((H, ROWS_BLK, 4 * DH), lambda b, i: (b, i, 0)),
        out_shape=jax.ShapeDtypeStruct((B * H, N, 4 * DH), jnp.float32),
    )(x, Wqk, bqk.reshape(1, D), Wv, bv.reshape(1, D), n1s, n1b)


def _k2_body(qkv_ref, r_ref, idx_ref):
    qk = qkv_ref[0, :, :DH]
    rot = jnp.dot(qk, r_ref[0], preferred_element_type=jnp.float32)
    lane16 = lax.broadcasted_iota(jnp.int32, (N, NB // 2), 1).astype(jnp.float32)
    mxp = jnp.max(rot, axis=1, keepdims=True)
    mxn = jnp.min(rot, axis=1, keepdims=True)
    ip = jnp.min(jnp.where(rot == mxp, lane16, float(NB)), axis=1, keepdims=True)
    iq = jnp.min(jnp.where(rot == mxn, lane16, float(NB)), axis=1, keepdims=True)
    bucket = jnp.where(mxp >= -mxn, ip, iq + float(NB // 2))
    lane = lax.broadcasted_iota(jnp.int32, (N, NB), 1).astype(jnp.float32)
    oh = (bucket == lane).astype(jnp.float32)

    gi = lax.broadcasted_iota(jnp.int32, (128, 128), 0)
    gj = lax.broadcasted_iota(jnp.int32, (128, 128), 1)
    t128 = (gj < gi).astype(jnp.float32)
    blocks = []
    prun = jnp.zeros((1, NB), jnp.float32)
    for g in range(N // 128):
        og = oh[g * 128:(g + 1) * 128, :]
        within = jnp.dot(t128, og, preferred_element_type=jnp.float32)
        blocks.append(within + prun)
        prun = prun + jnp.sum(og, axis=0, keepdims=True)
    ui = lax.broadcasted_iota(jnp.int32, (NB, NB), 0)
    uj = lax.broadcasted_iota(jnp.int32, (NB, NB), 1)
    u32 = (ui < uj).astype(jnp.float32)
    prefix = jnp.dot(prun, u32, preferred_element_type=jnp.float32, precision=lax.Precision.HIGHEST)
    cols = []
    for g in range(N // 128):
        ohg = oh[g * 128:(g + 1) * 128, :]
        cols.append(jnp.sum((blocks[g] + prefix) * ohg, axis=1, keepdims=True))
    rankmat = jnp.concatenate(cols, axis=1)
    bh = pl.program_id(0) * H + pl.program_id(1)
    idx_ref[...] = jnp.transpose(rankmat).astype(jnp.int32) + bh * N


def _k2(qk_t, R):
    return pl.pallas_call(
        _k2_body,
        grid=(B, H),
        in_specs=[
            pl.BlockSpec((1, N, 4 * DH), lambda b, h: (b * H + h, 0, 0)),
            pl.BlockSpec((1, DH, NB // 2), lambda b, h: (h, 0, 0)),
        ],
        out_specs=pl.BlockSpec((N // 128, 128), lambda b, h: (b * H + h, 0)),
        out_shape=jax.ShapeDtypeStruct((M // 128, 128), jnp.int32),
    )(qk_t, R)


def _sc_permute(qkv2d, idx2d):
    mesh = plsc.VectorSubcoreMesh(core_axis_name="c", subcore_axis_name="s")

    @functools.partial(
        pl.kernel, mesh=mesh,
        out_type=jax.ShapeDtypeStruct((M, 4 * DH), jnp.float32),
        scratch_types=[
            pltpu.VMEM((NJ, JROWS), jnp.int32),
            pltpu.VMEM((2 * SET, JROWS, 4 * DH), jnp.float32),
            pltpu.SemaphoreType.DMA,
            pltpu.SemaphoreType.DMA,
            pltpu.SemaphoreType.DMA,
            pltpu.SemaphoreType.DMA,
        ],
    )
    def k(qkv_hbm, idx_hbm, out_hbm, idx_v, rows_v, lsa, lsb, ssa, ssb):
        wid = lax.axis_index("s") * 2 + lax.axis_index("c")
        pltpu.sync_copy(idx_hbm.at[pl.ds(wid * NJ, NJ)], idx_v)
        nrounds = NJ // SET

        def load(r, bufs, sem, issue):
            for b in range(SET):
                cp = pltpu.make_async_copy(
                    qkv_hbm.at[pl.ds(wid * RPW + (r * SET + b) * JROWS, JROWS)],
                    rows_v.at[bufs + b], sem)
                cp.start() if issue else cp.wait()

        def scat(r, bufs, sem, issue):
            for b in range(SET):
                cp = pltpu.make_async_copy(
                    rows_v.at[bufs + b], out_hbm.at[idx_v.at[r * SET + b]], sem)
                cp.start() if issue else cp.wait()

        load(0, 0, lsa, True)

        def body(i, carry):
            r0 = i * 2
            load(r0 + 1, SET, lsb, True)
            load(r0, 0, lsa, False)
            scat(r0, 0, ssa, True)
            load(r0 + 1, SET, lsb, False)
            scat(r0 + 1, SET, ssb, True)
            scat(r0, 0, ssa, False)

            @pl.when(i < nrounds // 2 - 1)
            def _():
                load(r0 + 2, 0, lsa, True)

            scat(r0 + 1, SET, ssb, False)
            return carry

        lax.fori_loop(0, nrounds // 2, body, 0)

    return k(qkv2d, idx2d)


def _sc_unsort(sout2d, idx2d):
    mesh = plsc.VectorSubcoreMesh(core_axis_name="c", subcore_axis_name="s")

    @functools.partial(
        pl.kernel, mesh=mesh,
        out_type=jax.ShapeDtypeStruct((M, 4 * DH), jnp.float32),
        scratch_types=[
            pltpu.VMEM((NJ, JROWS), jnp.int32),
            pltpu.VMEM((2 * SET, JROWS, 4 * DH), jnp.float32),
            pltpu.SemaphoreType.DMA,
            pltpu.SemaphoreType.DMA,
            pltpu.SemaphoreType.DMA,
            pltpu.SemaphoreType.DMA,
        ],
    )
    def k(sout_hbm, idx_hbm, out_hbm, idx_v, rows_v, gsa, gsb, wsa, wsb):
        wid = lax.axis_index("s") * 2 + lax.axis_index("c")
        pltpu.sync_copy(idx_hbm.at[pl.ds(wid * NJ, NJ)], idx_v)
        nrounds = NJ // SET

        def gath(r, bufs, sem, issue):
            for b in range(SET):
                cp = pltpu.make_async_copy(
                    sout_hbm.at[idx_v.at[r * SET + b]], rows_v.at[bufs + b], sem)
                cp.start() if issue else cp.wait()

        def wr(r, bufs, sem, issue):
            for b in range(SET):
                cp = pltpu.make_async_copy(
                    rows_v.at[bufs + b],
                    out_hbm.at[pl.ds(wid * RPW + (r * SET + b) * JROWS, JROWS)],
                    sem)
                cp.start() if issue else cp.wait()

        gath(0, 0, gsa, True)

        def body(i, carry):
            r0 = i * 2
            gath(r0 + 1, SET, gsb, True)
            gath(r0, 0, gsa, False)
            wr(r0, 0, wsa, True)
            gath(r0 + 1, SET, gsb, False)
            wr(r0 + 1, SET, wsb, True)
            wr(r0, 0, wsa, False)

            @pl.when(i < nrounds // 2 - 1)
            def _():
                gath(r0 + 2, 0, gsa, True)

            wr(r0 + 1, SET, wsb, False)
            return carry

        lax.fori_loop(0, nrounds // 2, body, 0)

    return k(sout2d, idx2d)


BAND = 4 * CS


def _k3_body(sqkv_ref, out_ref):
    qk32 = sqkv_ref[0, :, :DH]
    v = sqkv_ref[0, :, DH:2 * DH].astype(jnp.bfloat16)
    qk = qk32.astype(jnp.bfloat16)
    nrm = jnp.sqrt(jnp.sum(qk32 * qk32, axis=1, keepdims=True))
    ck = (qk32 / (nrm + 1e-6)).astype(jnp.bfloat16)
    scale = 1.0 / jnp.sqrt(jnp.float32(DH))
    bi = lax.broadcasted_iota(jnp.int32, (BAND, BAND), 0)
    bj = lax.broadcasted_iota(jnp.int32, (BAND, BAND), 1)
    offblock = lax.div(bi, CS) != lax.div(bj, CS)
    for b0 in range(N // BAND):
        sl = slice(b0 * BAND, (b0 + 1) * BAND)
        scores = lax.dot_general(qk[sl, :], ck[sl, :], (((1,), (1,)), ((), ())),
                                 preferred_element_type=jnp.float32) * scale
        e = jnp.exp(jnp.where(offblock, -1e30, scores))
        attn = (e * (1.0 / jnp.sum(e, axis=1, keepdims=True))).astype(jnp.bfloat16)
        o = jnp.dot(attn, v[sl, :], preferred_element_type=jnp.float32)
        out_ref[0, sl, :DH] = o


def _k3(sqkv_t):
    return pl.pallas_call(
        _k3_body,
        grid=(B * H,),
        in_specs=[pl.BlockSpec((1, N, 4 * DH), lambda i: (i, 0, 0))],
        out_specs=pl.BlockSpec((1, N, 4 * DH), lambda i: (i, 0, 0)),
        out_shape=jax.ShapeDtypeStruct((B * H, N, 4 * DH), jnp.float32),
    )(sqkv_t)


def _k4_body(ao_ref, x_ref, wo_ref, bo_ref, n2s_ref, n2b_ref,
             wa_ref, ba_ref, wb_ref, bb_ref, wc_ref, bc_ref, out_ref):
    ao = jnp.concatenate(
        [ao_ref[hh, :, :DH] for hh in range(H)], axis=1).astype(jnp.bfloat16)
    o = jnp.dot(ao, wo_ref[...],
                preferred_element_type=jnp.float32) + bo_ref[...]
    h = x_ref[0] + o
    h2 = _ln(h, n2s_ref[...], n2b_ref[...]).astype(jnp.bfloat16)
    a = jnp.dot(h2, wa_ref[...], preferred_element_type=jnp.float32) + ba_ref[...]
    g = jnp.dot(h2, wb_ref[...], preferred_element_type=jnp.float32) + bb_ref[...]
    g = (a * jnp.maximum(g, 0.0)).astype(jnp.bfloat16)
    out_ref[0] = h + jnp.dot(g, wc_ref[...], preferred_element_type=jnp.float32) + bc_ref[...]


def _k4(ao2d, x2d, Wo, bo, n2s, n2b, Wa, ba, Wb, bb, Wc, bc):
    nb = N // ROWS_BLK
    return pl.pallas_call(
        _k4_body,
        grid=(B, nb),
        in_specs=[
            pl.BlockSpec((H, ROWS_BLK, 4 * DH), lambda b, i: (b, i, 0)),
            pl.BlockSpec((1, ROWS_BLK, D), lambda b, i: (b, i, 0)),
            pl.BlockSpec((D, D), lambda b, i: (0, 0)),
            pl.BlockSpec((1, D), lambda b, i: (0, 0)),
            pl.BlockSpec((1, D), lambda b, i: (0, 0)),
            pl.BlockSpec((1, D), lambda b, i: (0, 0)),
            pl.BlockSpec((D, F), lambda b, i: (0, 0)),
            pl.BlockSpec((1, F), lambda b, i: (0, 0)),
            pl.BlockSpec((D, F), lambda b, i: (0, 0)),
            pl.BlockSpec((1, F), lambda b, i: (0, 0)),
            pl.BlockSpec((F, D), lambda b, i: (0, 0)),
            pl.BlockSpec((1, D), lambda b, i: (0, 0)),
        ],
        out_specs=pl.BlockSpec((1, ROWS_BLK, D), lambda b, i: (b, i, 0)),
        out_shape=jax.ShapeDtypeStruct((B, N, D), jnp.float32),
    )(ao2d, x2d, Wo.astype(jnp.bfloat16), bo.reshape(1, D), n2s, n2b,
      Wa.astype(jnp.bfloat16), ba.reshape(1, F),
      Wb.astype(jnp.bfloat16), bb.reshape(1, F),
      Wc.astype(jnp.bfloat16), bc.reshape(1, D))


def kernel(x, Wqk, bqk, Wv, bv, Wo, bo, R, n1_scale, n1_bias,
           Wa, ba, Wb, bb, Wc, bc, n2_scale, n2_bias):
    qkv_t = _k1(x, Wqk, bqk, Wv, bv,
                n1_scale.reshape(1, D), n1_bias.reshape(1, D))

    idx2d = _k2(qkv_t, R)
    sqkv = _sc_permute(qkv_t.reshape(M, 4 * DH), idx2d)
    sout = _k3(sqkv.reshape(B * H, N, 4 * DH))
    out_t = _sc_unsort(sout.reshape(M, 4 * DH), idx2d)

    return _k4(out_t.reshape(B * H, N, 4 * DH), x,
               Wo, bo, n2_scale.reshape(1, D), n2_bias.reshape(1, D),
               Wa, ba, Wb, bb, Wc, bc)
